# baseline (speedup 1.0000x reference)
import functools

import jax
import jax.numpy as jnp
from jax import lax
from jax.experimental import pallas as pl
from jax.experimental.pallas import tpu as pltpu
from jax.experimental.pallas import tpu_sc as plsc

_NUM_CORES = 2
_NUM_SUBCORES = 16
_NUM_WORKERS = _NUM_CORES * _NUM_SUBCORES
_LANES = 16


@functools.lru_cache(maxsize=None)
def _build_relayout(vocab: int, dim: int):
    n_full = vocab // 128
    per_w = -(-n_full // _NUM_WORKERS)

    mesh = plsc.VectorSubcoreMesh(core_axis_name="c", subcore_axis_name="s")

    @functools.partial(
        pl.kernel,
        mesh=mesh,
        out_type=jax.ShapeDtypeStruct((vocab // 2, 2 * dim), jnp.float32),
        scratch_types=(
            [pltpu.VMEM((dim, 128), jnp.float32) for _ in range(2)]
            + [pltpu.VMEM((64, 2 * dim), jnp.float32) for _ in range(2)]
            + [pltpu.SemaphoreType.DMA for _ in range(4)]
        ),
        compiler_params=pltpu.CompilerParams(needs_layout_passes=False),
    )
    def relayout_kernel(wt_hbm, tail_hbm, w2_hbm, *scr):
        slab_v = scr[0:2]
        outb_v = scr[2:4]
        sem_s = scr[4:6]
        sem_o = scr[6:8]

        wid = lax.axis_index("s") * _NUM_CORES + lax.axis_index("c")
        blk0 = wid * per_w
        blk_end = jnp.minimum(blk0 + per_w, n_full)
        comp_iota = lax.iota(jnp.int32, _LANES)

        def slab_start(blk, b):
            pltpu.async_copy(wt_hbm.at[:, pl.ds(blk * 128, 128)], slab_v[b], sem_s[b])

        def slab_wait(blk, b):
            pltpu.make_async_copy(
                wt_hbm.at[:, pl.ds(blk * 128, 128)], slab_v[b], sem_s[b]
            ).wait()

        def transpose(b):
            vvecs = [2 * p0 + comp_iota for p0 in range(0, 64, 8)]
            pvecs = [
                p0 + lax.shift_right_logical(comp_iota, 1) for p0 in range(0, 64, 8)
            ]
            halfbase = lax.shift_left(lax.bitwise_and(comp_iota, 1), 6)

            def col_body(c, carry):
                for dc in (0, 32):
                    comp_vec = lax.bitwise_and(comp_iota + (c + dc), 63)
                    col_vec = halfbase + comp_vec
                    gathered = [
                        plsc.load_gather(slab_v[b], [comp_vec, vvecs[g]])
                        for g in range(8)
                    ]
                    for g in range(8):
                        plsc.store_scatter(outb_v[b], [pvecs[g], col_vec], gathered[g])
                return carry

            lax.fori_loop(0, 32, col_body, 0)

        def store_start(blk, b):
            pltpu.async_copy(outb_v[b], w2_hbm.at[pl.ds(blk * 64, 64), :], sem_o[b])

        def store_wait(blk, b):
            pltpu.make_async_copy(
                outb_v[b], w2_hbm.at[pl.ds(blk * 64, 64), :], sem_o[b]
            ).wait()

        @pl.when(blk0 < n_full)
        def _():
            slab_start(blk0, 0)

        def do_pair(t, carry):
            for b in range(2):
                blk = blk0 + 2 * t + b

                @pl.when(blk < blk_end)
                def _():
                    @pl.when(blk + 1 < blk_end)
                    def _():
                        slab_start(blk + 1, 1 - b)

                    slab_wait(blk, b)

                    @pl.when(blk - 2 >= blk0)
                    def _():
                        store_wait(blk - 2, b)

                    transpose(b)
                    store_start(blk, b)

            return carry

        lax.fori_loop(0, -(-per_w // 2), do_pair, 0)

        last_off = blk_end - 1 - blk0
        for b in range(2):
            blk_b = blk_end - 1 - lax.rem(last_off + 2 - b, 2)

            @pl.when(blk_b >= blk0)
            def _():
                store_wait(blk_b, b)

        @pl.when(wid == _NUM_WORKERS - 1)
        def _():
            n_tail_pairs = (vocab - n_full * 128) // 2
            pltpu.sync_copy(tail_hbm, outb_v[0].at[pl.ds(0, n_tail_pairs), :])
            pltpu.sync_copy(
                outb_v[0].at[pl.ds(0, n_tail_pairs), :],
                w2_hbm.at[pl.ds(n_full * 64, n_tail_pairs), :],
            )

    return relayout_kernel


@functools.lru_cache(maxsize=None)
def _build_lookup(batch: int, hist: int, dim: int, vocab: int):
    bw = batch // _NUM_WORKERS
    assert batch % (_NUM_WORKERS * _LANES) == 0
    n_groups = bw // _LANES

    mesh = plsc.VectorSubcoreMesh(core_axis_name="c", subcore_axis_name="s")

    @functools.partial(
        pl.kernel,
        mesh=mesh,
        out_type=jax.ShapeDtypeStruct((hist, dim, batch), jnp.float32),
        scratch_types=(
            [pltpu.VMEM((bw,), jnp.int32) for _ in range(2)]
            + [pltpu.VMEM((bw,), jnp.int32) for _ in range(2)]
            + [pltpu.VMEM((bw,), jnp.int32) for _ in range(2)]
            + [pltpu.VMEM((bw, 2 * dim), jnp.float32) for _ in range(2)]
            + [pltpu.VMEM((dim, bw), jnp.float32) for _ in range(2)]
            + [pltpu.SemaphoreType.DMA for _ in range(6)]
        ),
        compiler_params=pltpu.CompilerParams(needs_layout_passes=False),
    )
    def lookup_kernel(tok_hbm, w2_hbm, out_hbm, *scr):
        idx_v = scr[0:2]
        pp_v = scr[2:4]
        col_v = scr[4:6]
        pair_v = scr[6:8]
        outb_v = scr[8:10]
        sem_i = scr[10:12]
        sem_g = scr[12:14]
        sem_o = scr[14:16]

        wid = lax.axis_index("s") * _NUM_CORES + lax.axis_index("c")
        b0 = wid * bw
        lane_iota = lax.iota(jnp.int32, _LANES)

        def idx_start(h, b):
            pltpu.async_copy(tok_hbm.at[pl.ds(h * batch + b0, bw)], idx_v[b], sem_i[b])

        def idx_wait(h, b):
            pltpu.make_async_copy(
                tok_hbm.at[pl.ds(h * batch + b0, bw)], idx_v[b], sem_i[b]
            ).wait()

        def prep(b):
            for q in range(n_groups):
                sl = pl.ds(_LANES * q, _LANES)
                t = idx_v[b][sl]
                pp_v[b][sl] = lax.shift_right_logical(t, 1)
                col_v[b][sl] = lax.shift_left(lax.bitwise_and(t, 1), 6)

        def gather_start(b):
            pltpu.async_copy(w2_hbm.at[pp_v[b]], pair_v[b], sem_g[b])

        def gather_wait(b):
            pltpu.make_async_copy(w2_hbm.at[pp_v[b]], pair_v[b], sem_g[b]).wait()

        def transpose(b):
            j_vecs = [lane_iota + _LANES * q for q in range(n_groups)]
            col0s = [col_v[b][pl.ds(_LANES * q, _LANES)] for q in range(n_groups)]

            def col_body(c, carry):
                for dc in range(0, dim, dim // 4):
                    comp_vec = lax.bitwise_and(lane_iota + (c + dc), dim - 1)
                    gathered = [
                        plsc.load_gather(pair_v[b], [j_vecs[q], col0s[q] + comp_vec])
                        for q in range(n_groups)
                    ]
                    for q in range(n_groups):
                        plsc.store_scatter(
                            outb_v[b], [comp_vec, j_vecs[q]], gathered[q]
                        )
                return carry

            lax.fori_loop(0, dim // 4, col_body, 0)

        def store_start(h, b):
            pltpu.async_copy(outb_v[b], out_hbm.at[h, :, pl.ds(b0, bw)], sem_o[b])

        def store_wait(h, b):
            pltpu.make_async_copy(
                outb_v[b], out_hbm.at[h, :, pl.ds(b0, bw)], sem_o[b]
            ).wait()

        idx_start(0, 0)

        def do_group(t, carry):
            for b in range(2):
                h = 2 * t + b
                pb = 1 - b
                idx_wait(h, b)
                prep(b)
                gather_start(b)

                @pl.when(h + 1 < hist)
                def _():
                    idx_start(h + 1, pb)

                @pl.when(h >= 1)
                def _():
                    @pl.when(h >= 3)
                    def _():
                        store_wait(h - 3, pb)

                    gather_wait(pb)
                    transpose(pb)
                    store_start(h - 1, pb)

            return carry

        lax.fori_loop(0, hist // 2, do_group, 0)

        last = hist - 1
        lb = last % 2
        store_wait(last - 2, lb)
        gather_wait(lb)
        transpose(lb)
        store_start(last, lb)
        store_wait(last - 1, 1 - lb)
        store_wait(last, lb)

    return lookup_kernel


def kernel(token_ids, weight):
    batch, hist = token_ids.shape
    vocab, dim = weight.shape
    n_tail = vocab - (vocab // 128) * 128
    tail = weight[vocab - n_tail :, :].reshape(n_tail // 2, 2 * dim)
    w2 = _build_relayout(vocab, dim)(weight.T, tail)
    tok_flat = token_ids.T.reshape(-1).astype(jnp.int32)
    out_t = _build_lookup(batch, hist, dim, vocab)(tok_flat, w2)
    return out_t.transpose(2, 0, 1)


# speedup vs baseline: 1.0063x; 1.0063x over previous
import functools

import jax
import jax.numpy as jnp
from jax import lax
from jax.experimental import pallas as pl
from jax.experimental.pallas import tpu as pltpu
from jax.experimental.pallas import tpu_sc as plsc

_NUM_CORES = 2
_NUM_SUBCORES = 16
_NUM_WORKERS = _NUM_CORES * _NUM_SUBCORES
_LANES = 16


@functools.lru_cache(maxsize=None)
def _build_relayout(vocab: int, dim: int):
    n_full = vocab // 128
    per_w = -(-n_full // _NUM_WORKERS)

    mesh = plsc.VectorSubcoreMesh(core_axis_name="c", subcore_axis_name="s")

    @functools.partial(
        pl.kernel,
        mesh=mesh,
        out_type=jax.ShapeDtypeStruct((vocab // 2, 2 * dim), jnp.float32),
        scratch_types=(
            [pltpu.VMEM((dim, 128), jnp.float32) for _ in range(2)]
            + [pltpu.VMEM((64, 2 * dim), jnp.float32) for _ in range(2)]
            + [pltpu.SemaphoreType.DMA for _ in range(4)]
        ),
        compiler_params=pltpu.CompilerParams(needs_layout_passes=False),
    )
    def relayout_kernel(wt_hbm, tail_hbm, w2_hbm, *scr):
        slab_v = scr[0:2]
        outb_v = scr[2:4]
        sem_s = scr[4:6]
        sem_o = scr[6:8]

        wid = lax.axis_index("s") * _NUM_CORES + lax.axis_index("c")
        blk0 = wid * per_w
        blk_end = jnp.minimum(blk0 + per_w, n_full)
        comp_iota = lax.iota(jnp.int32, _LANES)

        def slab_start(blk, b):
            pltpu.async_copy(wt_hbm.at[:, pl.ds(blk * 128, 128)], slab_v[b], sem_s[b])

        def slab_wait(blk, b):
            pltpu.make_async_copy(
                wt_hbm.at[:, pl.ds(blk * 128, 128)], slab_v[b], sem_s[b]
            ).wait()

        def transpose(b):
            vvecs = [2 * p0 + comp_iota for p0 in range(0, 64, 8)]
            pvecs = [
                p0 + lax.shift_right_logical(comp_iota, 1) for p0 in range(0, 64, 8)
            ]
            halfbase = lax.shift_left(lax.bitwise_and(comp_iota, 1), 6)

            def col_body(c, carry):
                for dc in (0, 32):
                    comp_vec = lax.bitwise_and(comp_iota + (c + dc), 63)
                    col_vec = halfbase + comp_vec
                    gathered = [
                        plsc.load_gather(slab_v[b], [comp_vec, vvecs[g]])
                        for g in range(8)
                    ]
                    for g in range(8):
                        plsc.store_scatter(outb_v[b], [pvecs[g], col_vec], gathered[g])
                return carry

            lax.fori_loop(0, 32, col_body, 0)

        def store_start(blk, b):
            pltpu.async_copy(outb_v[b], w2_hbm.at[pl.ds(blk * 64, 64), :], sem_o[b])

        def store_wait(blk, b):
            pltpu.make_async_copy(
                outb_v[b], w2_hbm.at[pl.ds(blk * 64, 64), :], sem_o[b]
            ).wait()

        @pl.when(blk0 < n_full)
        def _():
            slab_start(blk0, 0)

        def do_pair(t, carry):
            for b in range(2):
                blk = blk0 + 2 * t + b

                @pl.when(blk < blk_end)
                def _():
                    @pl.when(blk + 1 < blk_end)
                    def _():
                        slab_start(blk + 1, 1 - b)

                    slab_wait(blk, b)

                    @pl.when(blk - 2 >= blk0)
                    def _():
                        store_wait(blk - 2, b)

                    transpose(b)
                    store_start(blk, b)

            return carry

        lax.fori_loop(0, -(-per_w // 2), do_pair, 0)

        last_off = blk_end - 1 - blk0
        for b in range(2):
            blk_b = blk_end - 1 - lax.rem(last_off + 2 - b, 2)

            @pl.when(blk_b >= blk0)
            def _():
                store_wait(blk_b, b)

        @pl.when(wid == _NUM_WORKERS - 1)
        def _():
            n_tail_pairs = (vocab - n_full * 128) // 2
            pltpu.sync_copy(tail_hbm, outb_v[0].at[pl.ds(0, n_tail_pairs), :])
            pltpu.sync_copy(
                outb_v[0].at[pl.ds(0, n_tail_pairs), :],
                w2_hbm.at[pl.ds(n_full * 64, n_tail_pairs), :],
            )

    return relayout_kernel


@functools.lru_cache(maxsize=None)
def _build_lookup(batch: int, hist: int, dim: int, vocab: int):
    bw = batch // _NUM_WORKERS
    assert batch % (_NUM_WORKERS * _LANES) == 0
    n_groups = bw // _LANES

    mesh = plsc.VectorSubcoreMesh(core_axis_name="c", subcore_axis_name="s")

    @functools.partial(
        pl.kernel,
        mesh=mesh,
        out_type=jax.ShapeDtypeStruct((hist, dim, batch), jnp.float32),
        scratch_types=(
            [pltpu.VMEM((bw,), jnp.int32) for _ in range(2)]
            + [pltpu.VMEM((bw,), jnp.int32) for _ in range(2)]
            + [pltpu.VMEM((bw,), jnp.int32) for _ in range(2)]
            + [pltpu.VMEM((bw, 2 * dim), jnp.float32) for _ in range(2)]
            + [pltpu.VMEM((dim, bw), jnp.float32) for _ in range(2)]
            + [pltpu.SemaphoreType.DMA for _ in range(6)]
        ),
        compiler_params=pltpu.CompilerParams(needs_layout_passes=False),
    )
    def lookup_kernel(tok_hbm, w2_hbm, out_hbm, *scr):
        idx_v = scr[0:2]
        pp_v = scr[2:4]
        col_v = scr[4:6]
        pair_v = scr[6:8]
        outb_v = scr[8:10]
        sem_i = scr[10:12]
        sem_g = scr[12:14]
        sem_o = scr[14:16]

        wid = lax.axis_index("s") * _NUM_CORES + lax.axis_index("c")
        b0 = wid * bw
        lane_iota = lax.iota(jnp.int32, _LANES)

        def idx_start(h, b):
            pltpu.async_copy(tok_hbm.at[pl.ds(h * batch + b0, bw)], idx_v[b], sem_i[b])

        def idx_wait(h, b):
            pltpu.make_async_copy(
                tok_hbm.at[pl.ds(h * batch + b0, bw)], idx_v[b], sem_i[b]
            ).wait()

        def prep(b):
            for q in range(n_groups):
                sl = pl.ds(_LANES * q, _LANES)
                t = idx_v[b][sl]
                pp_v[b][sl] = lax.shift_right_logical(t, 1)
                col_v[b][sl] = lax.shift_left(lax.bitwise_and(t, 1), 6)

        def gather_start(b):
            pltpu.async_copy(w2_hbm.at[pp_v[b]], pair_v[b], sem_g[b])

        def gather_wait(b):
            pltpu.make_async_copy(w2_hbm.at[pp_v[b]], pair_v[b], sem_g[b]).wait()

        def transpose(b):
            j_vecs = [lane_iota + _LANES * q for q in range(n_groups)]
            col0s = [col_v[b][pl.ds(_LANES * q, _LANES)] for q in range(n_groups)]

            def col_body(c, carry):
                for dc in range(0, dim, dim // 2):
                    comp_vec = lax.bitwise_and(lane_iota + (c + dc), dim - 1)
                    gathered = [
                        plsc.load_gather(pair_v[b], [j_vecs[q], col0s[q] + comp_vec])
                        for q in range(n_groups)
                    ]
                    for q in range(n_groups):
                        plsc.store_scatter(
                            outb_v[b], [comp_vec, j_vecs[q]], gathered[q]
                        )
                return carry

            lax.fori_loop(0, dim // 2, col_body, 0)

        def store_start(h, b):
            pltpu.async_copy(outb_v[b], out_hbm.at[h, :, pl.ds(b0, bw)], sem_o[b])

        def store_wait(h, b):
            pltpu.make_async_copy(
                outb_v[b], out_hbm.at[h, :, pl.ds(b0, bw)], sem_o[b]
            ).wait()

        idx_start(0, 0)

        def do_group(t, carry):
            for b in range(2):
                h = 2 * t + b
                pb = 1 - b
                idx_wait(h, b)
                prep(b)
                gather_start(b)

                @pl.when(h + 1 < hist)
                def _():
                    idx_start(h + 1, pb)

                @pl.when(h >= 1)
                def _():
                    @pl.when(h >= 3)
                    def _():
                        store_wait(h - 3, pb)

                    gather_wait(pb)
                    transpose(pb)
                    store_start(h - 1, pb)

            return carry

        lax.fori_loop(0, hist // 2, do_group, 0)

        last = hist - 1
        lb = last % 2
        store_wait(last - 2, lb)
        gather_wait(lb)
        transpose(lb)
        store_start(last, lb)
        store_wait(last - 1, 1 - lb)
        store_wait(last, lb)

    return lookup_kernel


def kernel(token_ids, weight):
    batch, hist = token_ids.shape
    vocab, dim = weight.shape
    n_tail = vocab - (vocab // 128) * 128
    tail = weight[vocab - n_tail :, :].reshape(n_tail // 2, 2 * dim)
    w2 = _build_relayout(vocab, dim)(weight.T, tail)
    tok_flat = token_ids.T.reshape(-1).astype(jnp.int32)
    out_t = _build_lookup(batch, hist, dim, vocab)(tok_flat, w2)
    return out_t.transpose(2, 0, 1)
